# baseline (speedup 1.0000x reference)
import functools

import jax
import jax.numpy as jnp
from jax import lax
from jax.experimental import pallas as pl
from jax.experimental.pallas import tpu as pltpu
from jax.experimental.pallas import tpu_sc as plsc

_B = 1024
_L = 50
_D = 768
_NC = 2
_NS = 16
_NW = _NC * _NS
_RPW = _B * _L // _NW
_LANES = 16
_DV = _D // _LANES
_CH = 64
_NCHUNK = _RPW // _CH
_GSC = _CH // _LANES
_REP = 8


def _make_kernel():
    mesh = plsc.VectorSubcoreMesh(core_axis_name="c", subcore_axis_name="s")

    @functools.partial(
        pl.kernel,
        mesh=mesh,
        out_type=[
            jax.ShapeDtypeStruct((_B * _L, _D), jnp.float32),
            jax.ShapeDtypeStruct((_B * _L,), jnp.int32),
        ],
        scratch_types=[
            pltpu.VMEM((_RPW,), jnp.int32),
            pltpu.VMEM((_B,), jnp.int32),
            pltpu.VMEM((_L * _REP + _REP,), jnp.int32),
            pltpu.VMEM((_RPW,), jnp.int32),
            pltpu.VMEM((_RPW,), jnp.int32),
            pltpu.VMEM((_CH, _D), jnp.float32),
            pltpu.VMEM((_CH, _D), jnp.float32),
            pltpu.VMEM((_LANES, _D), jnp.float32),
            pltpu.SemaphoreType.DMA,
            pltpu.SemaphoreType.DMA,
            pltpu.SemaphoreType.DMA,
            pltpu.SemaphoreType.DMA,
        ],
    )
    def k(tok_hbm, len_hbm, cnt_hbm, dst_hbm, table_hbm, out_hbm, mask_hbm,
          tok_v, len_v, cnt_v, dst_v, mask_v, rows0_v, rows1_v, zrows_v,
          g0, g1, w0, w1):
        rows = (rows0_v, rows1_v)
        gsem = (g0, g1)
        wsem = (w0, w1)

        wid = lax.axis_index("s") * _NC + lax.axis_index("c")
        row0 = wid * _RPW
        pltpu.sync_copy(tok_hbm.at[pl.ds(row0, _RPW)], tok_v)
        pltpu.sync_copy(len_hbm, len_v)
        pltpu.sync_copy(cnt_hbm, cnt_v)
        pltpu.sync_copy(dst_hbm.at[pl.ds(row0, _RPW)], dst_v)

        zeros = jnp.zeros((_LANES,), jnp.float32)

        def zfill(r, carry):
            for v in range(_DV):
                zrows_v[r, pl.ds(v * _LANES, _LANES)] = zeros
            return carry
        lax.fori_loop(0, _LANES, zfill, 0)

        def cnt_at(j):
            return cnt_v[pl.ds(j * _REP, _LANES)][0]

        def nvalid(c):
            p0 = row0 + c * _CH
            j = p0 // _B
            k0 = p0 - j * _B
            return jnp.clip(cnt_at(j) - k0, 0, _CH)

        def gather(c, b):
            return pltpu.make_async_copy(
                table_hbm.at[tok_v.at[pl.ds(c * _CH, _CH)]], rows[b], gsem[b])

        def scatter(c, q, src, sem):
            ivec = dst_v[pl.ds((c * _GSC + q) * _LANES, _LANES)]
            return pltpu.make_async_copy(src, out_hbm.at[ivec], sem)

        @pl.when(nvalid(0) > 0)
        def _():
            gather(0, 0).start()

        @pl.when(nvalid(1) > 0)
        def _():
            gather(1, 1).start()

        def mask_vec(n, carry):
            p0 = row0 + n * _LANES
            j = p0 // _B
            i0 = p0 - j * _B
            lv = len_v[pl.ds(i0, _LANES)]
            jv = jnp.zeros((_LANES,), jnp.int32) + j
            mask_v[pl.ds(n * _LANES, _LANES)] = jnp.where(
                jv < lv, jnp.ones((_LANES,), jnp.int32),
                jnp.zeros((_LANES,), jnp.int32))
            return carry
        lax.fori_loop(0, _RPW // _LANES, mask_vec, 0)
        pltpu.sync_copy(mask_v, mask_hbm.at[pl.ds(row0, _RPW)])

        def emit_writes(c, b):
            nv = nvalid(c)

            @pl.when(nv > 0)
            def _():
                def zrow(r, carry):
                    for v in range(_DV):
                        rows[b][r, pl.ds(v * _LANES, _LANES)] = zeros
                    return carry
                lax.fori_loop(nv, _CH, zrow, 0)
                for q in range(_GSC):
                    scatter(c, q, rows[b].at[pl.ds(q * _LANES, _LANES)],
                            wsem[b]).start()

            @pl.when(nv == 0)
            def _():
                for q in range(_GSC):
                    scatter(c, q, zrows_v, wsem[b]).start()

        def wait_writes(c, b):
            for q in range(_GSC):
                scatter(c, q, zrows_v, wsem[b]).wait()

        def step(c, b):
            @pl.when(nvalid(c) > 0)
            def _():
                gather(c, b).wait()
            emit_writes(c, b)
            o = 1 - b

            @pl.when(jnp.logical_and(c >= 1, c + 1 < _NCHUNK))
            def _():
                wait_writes(c - 1, o)
                nc = jnp.minimum(c + 1, _NCHUNK - 1)

                @pl.when(nvalid(nc) > 0)
                def _():
                    gather(nc, o).start()

        def pair(t, carry):
            step(t * 2, 0)
            step(t * 2 + 1, 1)
            return carry
        lax.fori_loop(0, _NCHUNK // 2, pair, 0)
        step(_NCHUNK - 1, (_NCHUNK - 1) % 2)

        wait_writes(_NCHUNK - 2, (_NCHUNK - 2) % 2)
        wait_writes(_NCHUNK - 1, (_NCHUNK - 1) % 2)

    return k


_sc_kernel = _make_kernel()


def kernel(batch_tokens, lengths, table):
    lengths = lengths.astype(jnp.int32)
    perm = jnp.argsort(-lengths).astype(jnp.int32)
    tok = batch_tokens.astype(jnp.int32)[perm].T.reshape(-1)
    cnt = jnp.sum(lengths[None, :] > jnp.arange(_L, dtype=jnp.int32)[:, None],
                  axis=1, dtype=jnp.int32)
    cnt_rep = jnp.concatenate(
        [jnp.repeat(cnt, _REP), jnp.zeros((_REP,), jnp.int32)])
    dst = (jnp.arange(_L, dtype=jnp.int32)[:, None] * _B + perm[None, :]
           ).reshape(-1)
    out_flat, mask_i = _sc_kernel(tok, lengths, cnt_rep, dst, table)
    embs = out_flat.reshape(_L, _B, _D).transpose(1, 0, 2)
    mask = (mask_i.reshape(_L, _B) != 0).T
    return embs, mask


# speedup vs baseline: 1.0266x; 1.0266x over previous
import functools

import jax
import jax.numpy as jnp
from jax import lax
from jax.experimental import pallas as pl
from jax.experimental.pallas import tpu as pltpu
from jax.experimental.pallas import tpu_sc as plsc

_B = 1024
_L = 50
_D = 768
_NC = 2
_NS = 16
_NW = _NC * _NS
_RPW = _B * _L // _NW
_LANES = 16
_DV = _D // _LANES
_CH = 64
_NCHUNK = _RPW // _CH
_GSC = _CH // _LANES
_REP = 8


def _make_kernel():
    mesh = plsc.VectorSubcoreMesh(core_axis_name="c", subcore_axis_name="s")

    @functools.partial(
        pl.kernel,
        mesh=mesh,
        out_type=[
            jax.ShapeDtypeStruct((_B * _L, _D), jnp.float32),
            jax.ShapeDtypeStruct((_B * _L,), jnp.int32),
        ],
        scratch_types=[
            pltpu.VMEM((_RPW,), jnp.int32),
            pltpu.VMEM((_B,), jnp.int32),
            pltpu.VMEM((_L * _REP + _REP,), jnp.int32),
            pltpu.VMEM((_RPW,), jnp.int32),
            pltpu.VMEM((_RPW,), jnp.int32),
            pltpu.VMEM((_CH, _D), jnp.float32),
            pltpu.VMEM((_CH, _D), jnp.float32),
            pltpu.VMEM((_LANES, _D), jnp.float32),
            pltpu.SemaphoreType.DMA,
            pltpu.SemaphoreType.DMA,
            pltpu.SemaphoreType.DMA,
            pltpu.SemaphoreType.DMA,
        ],
    )
    def k(tok_hbm, len_hbm, cnt_hbm, dst_hbm, table_hbm, out_hbm, mask_hbm,
          tok_v, len_v, cnt_v, dst_v, mask_v, rows0_v, rows1_v, zrows_v,
          g0, g1, w0, w1):
        rows = (rows0_v, rows1_v)
        gsem = (g0, g1)
        wsem = (w0, w1)

        wid = lax.axis_index("s") * _NC + lax.axis_index("c")
        row0 = wid * _RPW
        pltpu.sync_copy(tok_hbm.at[pl.ds(row0, _RPW)], tok_v)
        pltpu.sync_copy(len_hbm, len_v)
        pltpu.sync_copy(cnt_hbm, cnt_v)
        pltpu.sync_copy(dst_hbm.at[pl.ds(row0, _RPW)], dst_v)

        zeros = jnp.zeros((_LANES,), jnp.float32)

        def zfill(r, carry):
            for v in range(_DV):
                zrows_v[r, pl.ds(v * _LANES, _LANES)] = zeros
            return carry
        lax.fori_loop(0, _LANES, zfill, 0)

        def cnt_at(j):
            return cnt_v[pl.ds(j * _REP, _LANES)][0]

        def nvalid(c):
            p0 = row0 + c * _CH
            j = p0 // _B
            k0 = p0 - j * _B
            return jnp.clip(cnt_at(j) - k0, 0, _CH)

        def gather(c, b):
            return pltpu.make_async_copy(
                table_hbm.at[tok_v.at[pl.ds(c * _CH, _CH)]], rows[b], gsem[b])

        def scatter(c, q, src, sem):
            ivec = dst_v[pl.ds((c * _GSC + q) * _LANES, _LANES)]
            return pltpu.make_async_copy(src, out_hbm.at[ivec], sem)

        @pl.when(nvalid(0) > 0)
        def _():
            gather(0, 0).start()

        @pl.when(nvalid(1) > 0)
        def _():
            gather(1, 1).start()

        def mask_vec(n, carry):
            p0 = row0 + n * _LANES
            j = p0 // _B
            i0 = p0 - j * _B
            lv = len_v[pl.ds(i0, _LANES)]
            jv = jnp.zeros((_LANES,), jnp.int32) + j
            mask_v[pl.ds(n * _LANES, _LANES)] = jnp.where(
                jv < lv, jnp.ones((_LANES,), jnp.int32),
                jnp.zeros((_LANES,), jnp.int32))
            return carry
        lax.fori_loop(0, _RPW // _LANES, mask_vec, 0)
        pltpu.sync_copy(mask_v, mask_hbm.at[pl.ds(row0, _RPW)])

        def emit_writes(c, b):
            nv = nvalid(c)

            @pl.when(nv > 0)
            def _():
                def zrow(r, carry):
                    for v in range(_DV):
                        rows[b][r, pl.ds(v * _LANES, _LANES)] = zeros
                    return carry
                lax.fori_loop(nv, _CH, zrow, 0)
                for q in range(_GSC):
                    scatter(c, q, rows[b].at[pl.ds(q * _LANES, _LANES)],
                            wsem[b]).start()

            @pl.when(nv == 0)
            def _():
                for q in range(_GSC):
                    scatter(c, q, zrows_v, wsem[b]).start()

        def wait_writes(c, b):
            for q in range(_GSC):
                scatter(c, q, zrows_v, wsem[b]).wait()

        def step(c, b):
            @pl.when(nvalid(c) > 0)
            def _():
                gather(c, b).wait()
            o = 1 - b

            @pl.when(jnp.logical_and(c >= 1, c + 1 < _NCHUNK))
            def _():
                wait_writes(c - 1, o)
                nc = jnp.minimum(c + 1, _NCHUNK - 1)

                @pl.when(nvalid(nc) > 0)
                def _():
                    gather(nc, o).start()
            emit_writes(c, b)

        def pair(t, carry):
            step(t * 2, 0)
            step(t * 2 + 1, 1)
            return carry
        lax.fori_loop(0, _NCHUNK // 2, pair, 0)
        step(_NCHUNK - 1, (_NCHUNK - 1) % 2)

        wait_writes(_NCHUNK - 2, (_NCHUNK - 2) % 2)
        wait_writes(_NCHUNK - 1, (_NCHUNK - 1) % 2)

    return k


_sc_kernel = _make_kernel()


def kernel(batch_tokens, lengths, table):
    lengths = lengths.astype(jnp.int32)
    perm = jnp.argsort(-lengths).astype(jnp.int32)
    tok = batch_tokens.astype(jnp.int32)[perm].T.reshape(-1)
    cnt = jnp.sum(lengths[None, :] > jnp.arange(_L, dtype=jnp.int32)[:, None],
                  axis=1, dtype=jnp.int32)
    cnt_rep = jnp.concatenate(
        [jnp.repeat(cnt, _REP), jnp.zeros((_REP,), jnp.int32)])
    dst = (jnp.arange(_L, dtype=jnp.int32)[:, None] * _B + perm[None, :]
           ).reshape(-1)
    out_flat, mask_i = _sc_kernel(tok, lengths, cnt_rep, dst, table)
    embs = out_flat.reshape(_L, _B, _D).transpose(1, 0, 2)
    mask = (mask_i.reshape(_L, _B) != 0).T
    return embs, mask
